# baseline (speedup 1.0000x reference)
import functools

import jax
import jax.numpy as jnp
from jax import lax
from jax.experimental import pallas as pl
from jax.experimental.pallas import tpu as pltpu
from jax.experimental.pallas import tpu_sc as plsc

_NC = 2
_NS = 16
_NW = _NC * _NS
_D = 64
_L = 16


def _build_sc_kernel(n_items):
    per_w = n_items // _NW
    K = 256
    n_chunks = per_w // K
    half = (n_chunks - 1) // 2

    row_buf = pltpu.VMEM((K, _D), jnp.float32)
    idx_buf = pltpu.VMEM((K,), jnp.int32)

    @functools.partial(
        pl.kernel,
        out_type=jax.ShapeDtypeStruct((n_items,), jnp.float32),
        mesh=plsc.VectorSubcoreMesh(core_axis_name="c", subcore_axis_name="s"),
        compiler_params=pltpu.CompilerParams(
            needs_layout_passes=False, use_tc_tiling_on_sc=False),
        scratch_types=[
            idx_buf, idx_buf, idx_buf,
            idx_buf, idx_buf, idx_buf,
            row_buf, row_buf, row_buf,
            row_buf, row_buf, row_buf,
            pltpu.VMEM((K,), jnp.float32),
            pltpu.VMEM_SHARED((2000, _D), jnp.float32),
            pltpu.SemaphoreType.DMA, pltpu.SemaphoreType.DMA,
            pltpu.SemaphoreType.DMA, pltpu.SemaphoreType.DMA,
            pltpu.SemaphoreType.DMA, pltpu.SemaphoreType.DMA,
        ],
    )
    def sc_k(idx_hbm, ent_hbm, rel_hbm, out_hbm,
             hidx_a, tidx_a, ridx_a, hidx_b, tidx_b, ridx_b,
             hrow_a, trow_a, rrow_a, hrow_b, trow_b, rrow_b,
             out_v, rel_sh,
             sh_a, st_a, sr_a, sh_b, st_b, sr_b):
        wid = lax.axis_index("s") * _NC + lax.axis_index("c")
        wbase = wid * per_w
        lanes = lax.iota(jnp.int32, _L)

        @pl.when(lax.axis_index("s") == 0)
        def _():
            pltpu.sync_copy(rel_hbm, rel_sh)

        plsc.subcore_barrier()

        bufs_a = (hidx_a, tidx_a, ridx_a, hrow_a, trow_a, rrow_a,
                  sh_a, st_a, sr_a)
        bufs_b = (hidx_b, tidx_b, ridx_b, hrow_b, trow_b, rrow_b,
                  sh_b, st_b, sr_b)

        def issue(c, bufs):
            hidx_v, tidx_v, ridx_v, hrow_v, trow_v, rrow_v, sh, st, sr = bufs
            base = wbase + c * K
            pltpu.sync_copy(idx_hbm.at[pl.ds(base, K)], hidx_v)
            pltpu.sync_copy(idx_hbm.at[pl.ds(n_items + base, K)], tidx_v)
            pltpu.sync_copy(idx_hbm.at[pl.ds(2 * n_items + base, K)], ridx_v)
            pltpu.async_copy(ent_hbm.at[hidx_v], hrow_v, sh)
            pltpu.async_copy(ent_hbm.at[tidx_v], trow_v, st)
            pltpu.async_copy(rel_sh.at[ridx_v], rrow_v, sr)

        def wait_and_compute(c, bufs):
            hidx_v, tidx_v, ridx_v, hrow_v, trow_v, rrow_v, sh, st, sr = bufs
            base = wbase + c * K
            pltpu.make_async_copy(ent_hbm.at[hidx_v], hrow_v, sh).wait()
            pltpu.make_async_copy(ent_hbm.at[tidx_v], trow_v, st).wait()
            pltpu.make_async_copy(rel_sh.at[ridx_v], rrow_v, sr).wait()

            def group_body(g, carry):
                ids = g * _L + lanes
                acc = jnp.zeros((_L,), jnp.float32)
                for d in range(_D):
                    col = (lanes + d) & (_D - 1)
                    hv = plsc.load_gather(hrow_v, [ids, col])
                    tv = plsc.load_gather(trow_v, [ids, col])
                    rv = plsc.load_gather(rrow_v, [ids, col])
                    acc = acc + hv * tv * rv
                out_v[pl.ds(g * _L, _L)] = acc
                return carry

            lax.fori_loop(0, K // _L, group_body, 0)
            pltpu.sync_copy(out_v, out_hbm.at[pl.ds(base, K)])

        issue(0, bufs_a)

        def body2(c2, carry):
            c = 2 * c2
            issue(c + 1, bufs_b)
            wait_and_compute(c, bufs_a)
            issue(c + 2, bufs_a)
            wait_and_compute(c + 1, bufs_b)
            return carry

        lax.fori_loop(0, half, body2, 0)
        wait_and_compute(n_chunks - 1, bufs_a)

    return sc_k


def kernel(graph, batch, entity_emb, relation_emb):
    B, C, _ = batch.shape
    n = B * C
    n_ent = entity_emb.shape[0]
    planar = jnp.minimum(batch.transpose(2, 1, 0).reshape(3 * n), n_ent - 1) * 2
    ent = jnp.pad(entity_emb, ((0, 0), (0, _D))).reshape(2 * n_ent, _D)
    n_rel = relation_emb.shape[0]
    rel = jnp.pad(relation_emb, ((0, 0), (0, _D))).reshape(2 * n_rel, _D)
    sc_k = _build_sc_kernel(n)
    out = sc_k(planar, ent, rel)
    return out.reshape(C, B).transpose(1, 0)


# speedup vs baseline: 1.1362x; 1.1362x over previous
import functools

import jax
import jax.numpy as jnp
from jax import lax
from jax.experimental import pallas as pl
from jax.experimental.pallas import tpu as pltpu
from jax.experimental.pallas import tpu_sc as plsc

_NC = 2
_NS = 16
_NW = _NC * _NS
_D = 64
_L = 16


def _build_sc_kernel(n_items):
    per_w = n_items // _NW
    K = 256
    n_chunks = per_w // K
    half = (n_chunks - 1) // 2

    row_buf = pltpu.VMEM((K, _D), jnp.float32)
    idx_buf = pltpu.VMEM((K,), jnp.int32)

    @functools.partial(
        pl.kernel,
        out_type=jax.ShapeDtypeStruct((n_items,), jnp.float32),
        mesh=plsc.VectorSubcoreMesh(core_axis_name="c", subcore_axis_name="s"),
        compiler_params=pltpu.CompilerParams(
            needs_layout_passes=False, use_tc_tiling_on_sc=False,
            disable_bounds_checks=True),
        scratch_types=[
            idx_buf, idx_buf, idx_buf,
            idx_buf, idx_buf, idx_buf,
            row_buf, row_buf, row_buf,
            row_buf, row_buf, row_buf,
            pltpu.VMEM((K,), jnp.float32),
            pltpu.VMEM_SHARED((2000, _D), jnp.float32),
            pltpu.SemaphoreType.DMA, pltpu.SemaphoreType.DMA,
            pltpu.SemaphoreType.DMA, pltpu.SemaphoreType.DMA,
            pltpu.SemaphoreType.DMA, pltpu.SemaphoreType.DMA,
        ],
    )
    def sc_k(idx_hbm, ent_hbm, rel_hbm, out_hbm,
             hidx_a, tidx_a, ridx_a, hidx_b, tidx_b, ridx_b,
             hrow_a, trow_a, rrow_a, hrow_b, trow_b, rrow_b,
             out_v, rel_sh,
             sh_a, st_a, sr_a, sh_b, st_b, sr_b):
        wid = lax.axis_index("s") * _NC + lax.axis_index("c")
        wbase = wid * per_w
        lanes = lax.iota(jnp.int32, _L)

        @pl.when(lax.axis_index("s") == 0)
        def _():
            pltpu.sync_copy(rel_hbm, rel_sh)

        plsc.subcore_barrier()

        bufs_a = (hidx_a, tidx_a, ridx_a, hrow_a, trow_a, rrow_a,
                  sh_a, st_a, sr_a)
        bufs_b = (hidx_b, tidx_b, ridx_b, hrow_b, trow_b, rrow_b,
                  sh_b, st_b, sr_b)

        def issue(c, bufs):
            hidx_v, tidx_v, ridx_v, hrow_v, trow_v, rrow_v, sh, st, sr = bufs
            base = wbase + c * K
            pltpu.sync_copy(idx_hbm.at[pl.ds(base, K)], hidx_v)
            pltpu.sync_copy(idx_hbm.at[pl.ds(n_items + base, K)], tidx_v)
            pltpu.sync_copy(idx_hbm.at[pl.ds(2 * n_items + base, K)], ridx_v)
            pltpu.async_copy(ent_hbm.at[hidx_v], hrow_v, sh)
            pltpu.async_copy(ent_hbm.at[tidx_v], trow_v, st)
            pltpu.async_copy(rel_sh.at[ridx_v], rrow_v, sr)

        def wait_and_compute(c, bufs):
            hidx_v, tidx_v, ridx_v, hrow_v, trow_v, rrow_v, sh, st, sr = bufs
            base = wbase + c * K
            pltpu.make_async_copy(ent_hbm.at[hidx_v], hrow_v, sh).wait()
            pltpu.make_async_copy(ent_hbm.at[tidx_v], trow_v, st).wait()
            pltpu.make_async_copy(rel_sh.at[ridx_v], rrow_v, sr).wait()

            def group_body(g, carry):
                sv = jnp.zeros((_L,), jnp.float32)
                for u in range(_L):
                    i = g * _L + u
                    acc = None
                    for q in range(_D // _L):
                        hv = hrow_v[i, pl.ds(q * _L, _L)]
                        tv = trow_v[i, pl.ds(q * _L, _L)]
                        rv = rrow_v[i, pl.ds(q * _L, _L)]
                        p = hv * tv * rv
                        acc = p if acc is None else acc + p
                    sv = jnp.where(lanes == u, jnp.sum(acc), sv)
                out_v[pl.ds(g * _L, _L)] = sv
                return carry

            lax.fori_loop(0, K // _L, group_body, 0)
            pltpu.sync_copy(out_v, out_hbm.at[pl.ds(base, K)])

        issue(0, bufs_a)

        def body2(c2, carry):
            c = 2 * c2
            issue(c + 1, bufs_b)
            wait_and_compute(c, bufs_a)
            issue(c + 2, bufs_a)
            wait_and_compute(c + 1, bufs_b)
            return carry

        lax.fori_loop(0, half, body2, 0)
        wait_and_compute(n_chunks - 1, bufs_a)

    return sc_k


def kernel(graph, batch, entity_emb, relation_emb):
    B, C, _ = batch.shape
    n = B * C
    n_ent = entity_emb.shape[0]
    planar = jnp.minimum(batch.transpose(2, 1, 0).reshape(3 * n), n_ent - 1) * 2
    ent = jnp.pad(entity_emb, ((0, 0), (0, _D))).reshape(2 * n_ent, _D)
    n_rel = relation_emb.shape[0]
    rel = jnp.pad(relation_emb, ((0, 0), (0, _D))).reshape(2 * n_rel, _D)
    sc_k = _build_sc_kernel(n)
    out = sc_k(planar, ent, rel)
    return out.reshape(C, B).transpose(1, 0)
